# baseline (speedup 1.0000x reference)
import functools
import jax
import jax.numpy as jnp
from jax import lax
from jax.experimental import pallas as pl
from jax.experimental.pallas import tpu as pltpu
from jax.experimental.pallas import tpu_sc as plsc

_B = 16384
_D = 64
_H = 128
_L = 16
_W = 8

_info = plsc.get_sparse_core_info()
_NC, _NS = _info.num_cores, _info.num_subcores
_NW = _NC * _NS
_BPW = _B // _NW
_PH = 32
_NPHASE = _BPW // _PH

_sc_mesh = plsc.VectorSubcoreMesh(core_axis_name="c", subcore_axis_name="s")


@functools.partial(
    pl.kernel,
    out_type=(
        jax.ShapeDtypeStruct((_B, _W, _D), jnp.float32),
        jax.ShapeDtypeStruct((_B, _W, _D), jnp.float32),
    ),
    mesh=_sc_mesh,
    scratch_types=[
        pltpu.VMEM((_BPW,), jnp.int32),
        pltpu.VMEM((_BPW,), jnp.int32),
        pltpu.SMEM((_BPW,), jnp.int32),
        pltpu.SMEM((_BPW,), jnp.int32),
        pltpu.VMEM_SHARED((_NS, 2 * _BPW), jnp.int32),
        pltpu.VMEM((2, _PH, _W, _D), jnp.float32),
        pltpu.VMEM((2, _PH, _W, _D), jnp.float32),
        pltpu.SemaphoreType.DMA,
        pltpu.SemaphoreType.DMA,
        pltpu.SemaphoreType.DMA,
    ],
    compiler_params=pltpu.CompilerParams(use_tc_tiling_on_sc=False),
)
def _sc_gather(user_hbm, art_hbm, utab_hbm, atab_hbm, xu_hbm, xa_hbm,
               idx_uv, idx_av, idx_u, idx_a, sh_idx, slab_u, slab_a,
               sem_u, sem_a, sem_o):
    sid = lax.axis_index("s")
    wid = sid * _NC + lax.axis_index("c")
    base = wid * _BPW
    pltpu.sync_copy(user_hbm.at[pl.ds(base, _BPW)], idx_uv)
    pltpu.sync_copy(art_hbm.at[pl.ds(base, _BPW)], idx_av)
    pltpu.sync_copy(idx_uv, sh_idx.at[sid, pl.ds(0, _BPW)])
    pltpu.sync_copy(idx_av, sh_idx.at[sid, pl.ds(_BPW, _BPW)])
    pltpu.sync_copy(sh_idx.at[sid, pl.ds(0, _BPW)], idx_u)
    pltpu.sync_copy(sh_idx.at[sid, pl.ds(_BPW, _BPW)], idx_a)

    lane = lax.broadcasted_iota(jnp.int32, (_L,), 0)

    def extract(idx_ref, kk):
        v = idx_ref[pl.ds((kk // _L) * _L, _L)]
        return jnp.sum(jnp.where(lane == lax.rem(kk, _L), v, 0), axis=0)

    def phase(h, carry):
        p = lax.rem(h, 2)

        @pl.when(h >= 2)
        def _():
            pltpu.make_async_copy(
                slab_u.at[p], xu_hbm.at[pl.ds(base, _PH)], sem_o).wait()
            pltpu.make_async_copy(
                slab_a.at[p], xa_hbm.at[pl.ds(base, _PH)], sem_o).wait()

        def fetch(k, c):
            kk = h * _PH + k
            iu = idx_u[kk]
            ia = idx_a[kk]
            r0u = pl.multiple_of((iu >> 3) << 3, _W)
            r0a = pl.multiple_of((ia >> 3) << 3, _W)
            pltpu.async_copy(
                utab_hbm.at[pl.ds(r0u, _W), :], slab_u.at[p, k], sem_u)
            pltpu.async_copy(
                atab_hbm.at[pl.ds(r0a, _W), :], slab_a.at[p, k], sem_a)
            return c

        lax.fori_loop(0, _PH, fetch, 0)

        def drain(k, c):
            pltpu.make_async_copy(
                utab_hbm.at[pl.ds(0, _W), :], slab_u.at[0, 0], sem_u).wait()
            pltpu.make_async_copy(
                atab_hbm.at[pl.ds(0, _W), :], slab_a.at[0, 0], sem_a).wait()
            return c

        lax.fori_loop(0, _PH, drain, 0)

        pltpu.async_copy(
            slab_u.at[p], xu_hbm.at[pl.ds(base + h * _PH, _PH)], sem_o)
        pltpu.async_copy(
            slab_a.at[p], xa_hbm.at[pl.ds(base + h * _PH, _PH)], sem_o)
        return carry

    lax.fori_loop(0, _NPHASE, phase, 0)

    def drain_out(j, carry):
        pltpu.make_async_copy(
            slab_u.at[0], xu_hbm.at[pl.ds(base, _PH)], sem_o).wait()
        pltpu.make_async_copy(
            slab_a.at[0], xa_hbm.at[pl.ds(base, _PH)], sem_o).wait()
        return carry

    lax.fori_loop(0, 2, drain_out, 0)


_BLK = 2048


def _mlp_body(xu_ref, xa_ref, up_ref, ap_ref, w1a_ref, w1b_ref, b1_ref,
              w2_ref, b2_ref, out_ref):
    wsel = lax.broadcasted_iota(jnp.int32, (1, _W, 1), 1)
    xu = xu_ref[...].reshape(_BLK, _W, _D)
    xa = xa_ref[...].reshape(_BLK, _W, _D)
    um = (up_ref[...] & 7).reshape(_BLK, 1, 1) == wsel
    am = (ap_ref[...] & 7).reshape(_BLK, 1, 1) == wsel
    ue = jnp.sum(jnp.where(um, xu, 0.0), axis=1)
    ae = jnp.sum(jnp.where(am, xa, 0.0), axis=1)
    h = jnp.dot(ue, w1a_ref[...], preferred_element_type=jnp.float32)
    h += jnp.dot(ae, w1b_ref[...], preferred_element_type=jnp.float32)
    h = jnp.maximum(h + b1_ref[...], 0.0)
    o = jnp.dot(h, w2_ref[...], preferred_element_type=jnp.float32)
    out_ref[...] = jax.nn.sigmoid(o + b2_ref[...])


_mlp = pl.pallas_call(
    _mlp_body,
    grid=(_B // _BLK,),
    in_specs=[
        pl.BlockSpec((_BLK, _W * _D), lambda i: (i, 0)),
        pl.BlockSpec((_BLK, _W * _D), lambda i: (i, 0)),
        pl.BlockSpec((_BLK, 1), lambda i: (i, 0)),
        pl.BlockSpec((_BLK, 1), lambda i: (i, 0)),
        pl.BlockSpec((_D, _H), lambda i: (0, 0)),
        pl.BlockSpec((_D, _H), lambda i: (0, 0)),
        pl.BlockSpec((1, _H), lambda i: (0, 0)),
        pl.BlockSpec((_H, 1), lambda i: (0, 0)),
        pl.BlockSpec((1, 1), lambda i: (0, 0)),
    ],
    out_specs=pl.BlockSpec((_BLK, 1), lambda i: (i, 0)),
    out_shape=jax.ShapeDtypeStruct((_B, 1), jnp.float32),
)


@jax.jit
def kernel(user, artwork, user_table, artwork_table, W1, b1, W2, b2):
    xu, xa = _sc_gather(user, artwork, user_table, artwork_table)
    return _mlp(xu.reshape(_B, _W * _D), xa.reshape(_B, _W * _D),
                user.reshape(_B, 1), artwork.reshape(_B, 1),
                W1[:, :_D].T, W1[:, _D:].T, b1.reshape(1, _H), W2.T,
                b2.reshape(1, 1))


# speedup vs baseline: 1.4724x; 1.4724x over previous
import functools
import jax
import jax.numpy as jnp
from jax import lax
from jax.experimental import pallas as pl
from jax.experimental.pallas import tpu as pltpu
from jax.experimental.pallas import tpu_sc as plsc

_B = 16384
_D = 64
_H = 128

_info = plsc.get_sparse_core_info()
_NC, _NS = _info.num_cores, _info.num_subcores
_NW = _NC * _NS
_BPW = _B // _NW
_NSTR = 16
_CH = _BPW // _NSTR

_sc_mesh = plsc.VectorSubcoreMesh(core_axis_name="c", subcore_axis_name="s")


@functools.partial(
    pl.kernel,
    out_type=(
        jax.ShapeDtypeStruct((_B, _D), jnp.float32),
        jax.ShapeDtypeStruct((_B, _D), jnp.float32),
    ),
    mesh=_sc_mesh,
    scratch_types=[
        pltpu.VMEM((_BPW,), jnp.int32),
        pltpu.VMEM((_BPW,), jnp.int32),
        pltpu.VMEM((_BPW, _D), jnp.float32),
        pltpu.VMEM((_BPW, _D), jnp.float32),
        pltpu.SemaphoreType.DMA,
        pltpu.SemaphoreType.DMA,
    ],
    compiler_params=pltpu.CompilerParams(use_tc_tiling_on_sc=False),
)
def _sc_gather(user_hbm, art_hbm, utab_hbm, atab_hbm, ue_hbm, ae_hbm,
               idx_u, idx_a, rows_u, rows_a, sem_u, sem_a):
    wid = lax.axis_index("s") * _NC + lax.axis_index("c")
    base = wid * _BPW
    pltpu.sync_copy(user_hbm.at[pl.ds(base, _BPW)], idx_u)
    pltpu.sync_copy(art_hbm.at[pl.ds(base, _BPW)], idx_a)

    def issue(j, carry):
        o = j * _CH
        pltpu.async_copy(
            utab_hbm.at[idx_u.at[pl.ds(o, _CH)]], rows_u.at[pl.ds(o, _CH)],
            sem_u)
        pltpu.async_copy(
            atab_hbm.at[idx_a.at[pl.ds(o, _CH)]], rows_a.at[pl.ds(o, _CH)],
            sem_a)
        return carry

    lax.fori_loop(0, _NSTR, issue, 0)

    def drain(j, carry):
        pltpu.make_async_copy(
            utab_hbm.at[pl.ds(0, _CH)], rows_u.at[pl.ds(0, _CH)], sem_u).wait()
        pltpu.make_async_copy(
            atab_hbm.at[pl.ds(0, _CH)], rows_a.at[pl.ds(0, _CH)], sem_a).wait()
        return carry

    lax.fori_loop(0, _NSTR, drain, 0)

    pltpu.sync_copy(rows_u, ue_hbm.at[pl.ds(base, _BPW)])
    pltpu.sync_copy(rows_a, ae_hbm.at[pl.ds(base, _BPW)])


_BLK = 2048


def _mlp_body(ue_ref, ae_ref, w1a_ref, w1b_ref, b1_ref, w2_ref, b2_ref, out_ref):
    h = jnp.dot(ue_ref[...], w1a_ref[...], preferred_element_type=jnp.float32)
    h += jnp.dot(ae_ref[...], w1b_ref[...], preferred_element_type=jnp.float32)
    h = jnp.maximum(h + b1_ref[...], 0.0)
    o = jnp.dot(h, w2_ref[...], preferred_element_type=jnp.float32)
    out_ref[...] = jax.nn.sigmoid(o + b2_ref[...])


_mlp = pl.pallas_call(
    _mlp_body,
    grid=(_B // _BLK,),
    in_specs=[
        pl.BlockSpec((_BLK, _D), lambda i: (i, 0)),
        pl.BlockSpec((_BLK, _D), lambda i: (i, 0)),
        pl.BlockSpec((_D, _H), lambda i: (0, 0)),
        pl.BlockSpec((_D, _H), lambda i: (0, 0)),
        pl.BlockSpec((1, _H), lambda i: (0, 0)),
        pl.BlockSpec((_H, 1), lambda i: (0, 0)),
        pl.BlockSpec((1, 1), lambda i: (0, 0)),
    ],
    out_specs=pl.BlockSpec((_BLK, 1), lambda i: (i, 0)),
    out_shape=jax.ShapeDtypeStruct((_B, 1), jnp.float32),
)


@jax.jit
def kernel(user, artwork, user_table, artwork_table, W1, b1, W2, b2):
    ue, ae = _sc_gather(user, artwork, user_table, artwork_table)
    w1a = W1[:, :_D].T
    w1b = W1[:, _D:].T
    return _mlp(ue, ae, w1a, w1b, b1.reshape(1, _H), W2.T, b2.reshape(1, 1))
